# baseline (speedup 1.0000x reference)
import jax
import jax.numpy as jnp
from jax import lax
from jax.experimental import pallas as pl
from jax.experimental.pallas import tpu as pltpu
from jax.experimental.pallas import tpu_sc as plsc

B, H, W = 4, 1024, 1920
N = H * W
NC, NS, L = 2, 16, 16
ROWS_PER_TILE = H // NS
CHUNK = ROWS_PER_TILE * W
WIN = 384
SUBW = W // WIN
NWIN = ROWS_PER_TILE * SUBW
NJ = WIN // 128
GROUPS = WIN // L
ZCHUNK = CHUNK // 8
BATCH_PER_CORE = B // NC
SCAT_BYTES = 4 * NJ * 128 * 4


def _splat_body(fx_hbm, fy_hbm, w_hbm, zeros_hbm, cnt_hbm, o0_hbm, o1_hbm,
                acc, fxb, fyb, wb, valb, idxb, drainb, sem_in, sem_sc):
    c = lax.axis_index("c")
    t = lax.axis_index("s")
    toff = t * CHUNK
    ids = lax.iota(jnp.int32, L)

    def in_bufs(p):
        return fxb.at[p], fyb.at[p], wb.at[p]

    def fire_inputs(b, w, p):
        row = t * ROWS_PER_TILE + w // SUBW
        base = b * N + row * W + (w % SUBW) * WIN
        fxw, fyw, ww = in_bufs(p)
        pltpu.async_copy(fx_hbm.at[pl.ds(base, WIN)], fxw, sem_in)
        pltpu.async_copy(fy_hbm.at[pl.ds(base, WIN)], fyw, sem_in)
        pltpu.async_copy(w_hbm.at[pl.ds(base, WIN)], ww, sem_in)

    def wait_inputs(p):
        fxw, fyw, ww = in_bufs(p)
        pltpu.make_async_copy(fx_hbm.at[pl.ds(0, WIN)], fxw, sem_in).wait()
        pltpu.make_async_copy(fy_hbm.at[pl.ds(0, WIN)], fyw, sem_in).wait()
        pltpu.make_async_copy(w_hbm.at[pl.ds(0, WIN)], ww, sem_in).wait()

    def drain_scatters():
        pltpu.make_async_copy(
            zeros_hbm.at[pl.ds(0, 4 * WIN)], drainb, sem_sc).wait()

    def do_round(b, a_static, out_ref):
        def zero_body(z, carry):
            pltpu.sync_copy(zeros_hbm, acc.at[pl.ds(toff + z * ZCHUNK, ZCHUNK)])
            return carry

        lax.fori_loop(0, CHUNK // ZCHUNK, zero_body, 0)
        plsc.subcore_barrier()

        fire_inputs(b, 0, 0)

        def win_loop(w, carry):
            p = lax.rem(w, 2)
            wait_inputs(p)

            @pl.when(w + 1 < NWIN)
            def _():
                fire_inputs(b, w + 1, 1 - p)

            @pl.when(w >= 2)
            def _():
                drain_scatters()

            row = t * ROWS_PER_TILE + w // SUBW
            yf = row.astype(jnp.float32)
            x0 = (w % SUBW) * WIN
            fxw, fyw, ww = in_bufs(p)
            valw = valb.at[p]
            idxw = idxb.at[p]

            def group(g, gcarry):
                s = g * L
                fx = fxw[pl.ds(s, L)]
                fy = fyw[pl.ds(s, L)]
                wv = ww[pl.ds(s, L)]
                xf = (x0 + s + ids).astype(jnp.float32)
                x2 = xf + fx
                y2 = yf + fy
                valid = ((x2 >= 0.0) & (y2 >= 0.0)
                         & (x2 <= float(W - 1)) & (y2 <= float(H - 1)))
                x2c = jnp.clip(x2, 0.0, float(W - 1))
                y2c = jnp.clip(y2, 0.0, float(H - 1))
                ixL = x2c.astype(jnp.int32)
                iyT = y2c.astype(jnp.int32)
                dx = jnp.minimum(ixL + 1, W - 1) - ixL
                dW = (jnp.minimum(iyT + 1, H - 1) - iyT) * W
                i0 = iyT * W + ixL
                wd = jnp.where(valid, wv, 0.0)
                if a_static == 0:
                    v = wd
                elif a_static == 1:
                    v = -fx * wd
                else:
                    v = -fy * wd
                valw[pl.ds(s, L)] = v
                j = g // 8
                col = (g % 8) * L
                idxw[j, pl.ds(col, L)] = i0
                idxw[NJ + j, pl.ds(col, L)] = i0 + dx
                idxw[2 * NJ + j, pl.ds(col, L)] = i0 + dW
                idxw[3 * NJ + j, pl.ds(col, L)] = i0 + dW + dx
                return gcarry

            lax.fori_loop(0, GROUPS, group, 0)

            def scat(j, scarry):
                src = valw.at[pl.ds(j * 128, 128)]
                pltpu.async_copy(src, acc.at[idxw.at[j]], sem_sc, add=True)
                pltpu.async_copy(src, acc.at[idxw.at[NJ + j]], sem_sc,
                                 add=True)
                pltpu.async_copy(src, acc.at[idxw.at[2 * NJ + j]], sem_sc,
                                 add=True)
                pltpu.async_copy(src, acc.at[idxw.at[3 * NJ + j]], sem_sc,
                                 add=True)
                return scarry

            lax.fori_loop(0, NJ, scat, 0)
            return carry

        lax.fori_loop(0, NWIN, win_loop, 0)
        drain_scatters()
        drain_scatters()
        plsc.subcore_barrier()

        def copyout(z, carry):
            pltpu.sync_copy(
                acc.at[pl.ds(toff + z * ZCHUNK, ZCHUNK)],
                out_ref.at[pl.ds(b * N + toff + z * ZCHUNK, ZCHUNK)])
            return carry

        lax.fori_loop(0, CHUNK // ZCHUNK, copyout, 0)

    for a_static, out_ref in ((0, cnt_hbm), (1, o0_hbm), (2, o1_hbm)):
        def rounds(bi, carry, _a=a_static, _o=out_ref):
            do_round(c * BATCH_PER_CORE + bi, _a, _o)
            return carry

        lax.fori_loop(0, BATCH_PER_CORE, rounds, 0)


HC = 128


def _norm_body(cnt_ref, o0_ref, o1_ref, out_ref):
    cv = cnt_ref[0]
    m = cv > 0.0
    d = jnp.where(m, cv, 1.0)
    o0 = o0_ref[0]
    o1 = o1_ref[0]
    out_ref[0, 0] = jnp.where(m, o0 / d, o0)
    out_ref[0, 1] = jnp.where(m, o1 / d, o1)


def _normalize(cnt, o0, o1):
    spec3 = pl.BlockSpec((1, HC, W), lambda b, h: (b, h, 0))
    return pl.pallas_call(
        _norm_body,
        grid=(B, H // HC),
        in_specs=[spec3, spec3, spec3],
        out_specs=pl.BlockSpec((1, 2, HC, W), lambda b, h: (b, 0, h, 0)),
        out_shape=jax.ShapeDtypeStruct((B, 2, H, W), jnp.float32),
    )(cnt, o0, o1)


@jax.jit
def kernel(input1, input2):
    fx = input1[:, 0].reshape(-1)
    fy = input1[:, 1].reshape(-1)
    w = input2[:, 0].reshape(-1)
    zeros = jnp.zeros((ZCHUNK,), jnp.float32)
    mesh = plsc.VectorSubcoreMesh(core_axis_name="c", subcore_axis_name="s")
    cnt, o0, o1 = pl.kernel(
        _splat_body,
        out_type=[jax.ShapeDtypeStruct((B * N,), jnp.float32)] * 3,
        mesh=mesh,
        scratch_types=[
            pltpu.VMEM_SHARED((N,), jnp.float32),
            pltpu.VMEM((2, WIN), jnp.float32),
            pltpu.VMEM((2, WIN), jnp.float32),
            pltpu.VMEM((2, WIN), jnp.float32),
            pltpu.VMEM((2, WIN), jnp.float32),
            pltpu.VMEM((2, 4 * NJ, 128), jnp.int32),
            pltpu.VMEM((4 * WIN,), jnp.float32),
            pltpu.SemaphoreType.DMA,
            pltpu.SemaphoreType.DMA,
        ],
    )(fx, fy, w, zeros)
    return _normalize(cnt.reshape(B, H, W), o0.reshape(B, H, W),
                      o1.reshape(B, H, W))
